# Initial kernel scaffold; baseline (speedup 1.0000x reference)
import jax
import jax.numpy as jnp
from jax.experimental import pallas as pl


def kernel(images, features, W_conv, b_conv, W_cls, b_cls, W_loc, b_loc):
    raise NotImplementedError("write your pallas kernel here")


# speedup vs baseline: 3.9302x; 3.9302x over previous
import numpy as np
import jax
import jax.numpy as jnp
from jax.experimental import pallas as pl
from jax.experimental.pallas import tpu as pltpu

_STRIDE = 16
_SCALES = np.array([128.0, 256.0, 512.0], dtype=np.float32)
_RATIOS = np.array([0.5, 1.0, 2.0], dtype=np.float32)
_PRE = 1000
_POST = 300
_NMS_THR = 0.7
_MIN_SIZE = 16.0
_FH = 50
_PG = 52
_ROWS = 2598
_NPAD = 1024


def _anchor_grids():
    ws = (_SCALES[None, :] * np.sqrt(1.0 / _RATIOS)[:, None]).reshape(-1)
    hs = (_SCALES[None, :] * np.sqrt(_RATIOS)[:, None]).reshape(-1)
    r = np.arange(53, 53 + _ROWS)
    gy, gx = r // _PG, r % _PG
    cx = ((gx - 1) + 0.5) * _STRIDE
    cy = ((gy - 1) + 0.5) * _STRIDE
    out = np.empty((4, _ROWS, 9), dtype=np.float32)
    out[0] = cx[:, None]
    out[1] = cy[:, None]
    out[2] = ws[None, :]
    out[3] = hs[None, :]
    return jnp.asarray(out)


def _head_body(x_ref, w9_ref, bconv_ref, wcls_ref, bcls_ref, wd_ref, bd_ref,
               anc_ref, ms_ref, b0_ref, b1_ref, b2_ref, b3_ref, *, img_h, img_w):
    acc = jnp.zeros((_ROWS, 256), jnp.float32)
    for dy in (-1, 0, 1):
        for dx in (-1, 0, 1):
            k = (dy + 1) * 3 + (dx + 1)
            start = 53 + dy * _PG + dx
            xs = x_ref[0, pl.ds(start, _ROWS), :]
            acc += jnp.dot(xs, w9_ref[k], preferred_element_type=jnp.float32)
    h = jnp.maximum(acc + bconv_ref[:, :], 0.0)

    logits = jnp.dot(h, wcls_ref[:, :], preferred_element_type=jnp.float32) + bcls_ref[:, :]
    score = 1.0 / (1.0 + jnp.exp(-logits))

    dxv = jnp.dot(h, wd_ref[0], preferred_element_type=jnp.float32) + bd_ref[0]
    dyv = jnp.dot(h, wd_ref[1], preferred_element_type=jnp.float32) + bd_ref[1]
    dwv = jnp.dot(h, wd_ref[2], preferred_element_type=jnp.float32) + bd_ref[2]
    dhv = jnp.dot(h, wd_ref[3], preferred_element_type=jnp.float32) + bd_ref[3]
    dwv = jnp.minimum(dwv, 4.135)
    dhv = jnp.minimum(dhv, 4.135)

    cxa = anc_ref[0]
    cya = anc_ref[1]
    wa = anc_ref[2]
    ha = anc_ref[3]
    pcx = dxv * wa + cxa
    pcy = dyv * ha + cya
    pw = jnp.exp(dwv) * wa
    ph = jnp.exp(dhv) * ha

    c0 = jnp.clip(pcx - 0.5 * pw, 0.0, img_h)
    c1 = jnp.clip(pcy - 0.5 * ph, 0.0, img_w)
    c2 = jnp.clip(pcx + 0.5 * pw, 0.0, img_h)
    c3 = jnp.clip(pcy + 0.5 * ph, 0.0, img_w)
    hsz = c2 - c0
    wsz = c3 - c1
    valid = (hsz >= _MIN_SIZE) & (wsz >= _MIN_SIZE) & (score >= 0.0)
    ms_ref[0] = jnp.where(valid, score, -1e9)
    b0_ref[0] = c0
    b1_ref[0] = c1
    b2_ref[0] = c2
    b3_ref[0] = c3


def _nms_body(rc_ref, rt_ref, sc_ref, sr_ref, ob_ref, os_ref, sup_ref):
    rc = rc_ref[0]
    rt = rt_ref[0]
    x1c, y1c, x2c, y2c = rc[:, 0:1], rc[:, 1:2], rc[:, 2:3], rc[:, 3:4]
    x1r, y1r, x2r, y2r = rt[0:1, :], rt[1:2, :], rt[2:3, :], rt[3:4, :]

    area_c = (x2c - x1c) * (y2c - y1c)
    area_r = (x2r - x1r) * (y2r - y1r)
    iw = jnp.maximum(jnp.minimum(x2c, x2r) - jnp.maximum(x1c, x1r), 0.0)
    ih = jnp.maximum(jnp.minimum(y2c, y2r) - jnp.maximum(y1c, y1r), 0.0)
    inter = iw * ih
    iou = inter / (area_c + area_r - inter + 1e-9)

    ii = jax.lax.broadcasted_iota(jnp.int32, (_NPAD, _NPAD), 0)
    jj = jax.lax.broadcasted_iota(jnp.int32, (_NPAD, _NPAD), 1)
    sup_ref[:, :] = jnp.where((iou > _NMS_THR) & (jj > ii), 1.0, 0.0)
    lane = jax.lax.broadcasted_iota(jnp.int32, (1, _NPAD), 1)

    def body(i, keep):
        row = sup_ref[pl.ds(i, 1), :]
        ki = jnp.sum(keep * jnp.where(lane == i, 1.0, 0.0))
        return keep * (1.0 - ki * row)

    keep = jax.lax.fori_loop(0, _PRE, body, jnp.ones((1, _NPAD), jnp.float32))

    keep = keep * jnp.where(sr_ref[0] > -1e8, 1.0, 0.0)
    lt = jnp.where(ii <= jj, 1.0, 0.0)
    pos = jnp.dot(keep, lt, preferred_element_type=jnp.float32)
    kk = jax.lax.broadcasted_iota(jnp.int32, (304, _NPAD), 0).astype(jnp.float32) + 1.0
    m = jnp.where(pos == kk, 1.0, 0.0) * keep
    ob_ref[0] = jnp.dot(m, rc, preferred_element_type=jnp.float32)
    os_ref[0] = jnp.dot(m, sc_ref[0], preferred_element_type=jnp.float32)


def _unpad(a):
    full = jnp.pad(a, ((0, 0), (53, 53), (0, 0)))
    g = full.reshape(2, _PG, _PG, 9)[:, 1:51, 1:51, :]
    return g.reshape(2, _FH * _FH * 9)


def kernel(images, features, W_conv, b_conv, W_cls, b_cls, W_loc, b_loc):
    img_h = float(images.shape[2])
    img_w = float(images.shape[3])
    n = features.shape[0]

    xt = jnp.transpose(features, (0, 2, 3, 1))
    xp = jnp.pad(xt, ((0, 0), (1, 1), (1, 1), (0, 0)))
    x = xp.reshape(n, _PG * _PG, 256)

    w9 = jnp.transpose(W_conv, (2, 3, 1, 0)).reshape(9, 256, 256)
    wcls = W_cls[:, :, 0, 0].T
    wloc = W_loc[:, :, 0, 0].T
    wd = jnp.stack([wloc[:, c::4] for c in range(4)])
    bd = jnp.stack([b_loc[c::4] for c in range(4)])[:, None, :]
    anc = _anchor_grids()

    import functools
    head = pl.pallas_call(
        functools.partial(_head_body, img_h=img_h, img_w=img_w),
        grid=(n,),
        in_specs=[
            pl.BlockSpec((1, _PG * _PG, 256), lambda i: (i, 0, 0)),
            pl.BlockSpec((9, 256, 256), lambda i: (0, 0, 0)),
            pl.BlockSpec((1, 256), lambda i: (0, 0)),
            pl.BlockSpec((256, 9), lambda i: (0, 0)),
            pl.BlockSpec((1, 9), lambda i: (0, 0)),
            pl.BlockSpec((4, 256, 9), lambda i: (0, 0, 0)),
            pl.BlockSpec((4, 1, 9), lambda i: (0, 0, 0)),
            pl.BlockSpec((4, _ROWS, 9), lambda i: (0, 0, 0)),
        ],
        out_specs=[pl.BlockSpec((1, _ROWS, 9), lambda i: (i, 0, 0))] * 5,
        out_shape=[jax.ShapeDtypeStruct((n, _ROWS, 9), jnp.float32)] * 5,
    )
    ms, c0, c1, c2, c3 = head(x, w9, b_conv.reshape(1, 256), wcls,
                              b_cls.reshape(1, 9), wd, bd, anc)

    masked = _unpad(ms)
    boxes_flat = jnp.stack([_unpad(c0), _unpad(c1), _unpad(c2), _unpad(c3)], axis=-1)
    top_scores, order = jax.lax.top_k(masked, _PRE)
    top_rois = jnp.take_along_axis(boxes_flat, order[..., None], axis=1)

    rois_p = jnp.pad(top_rois, ((0, 0), (0, _NPAD - _PRE), (0, 0)))
    sp = jnp.pad(top_scores, ((0, 0), (0, _NPAD - _PRE)), constant_values=-1e9)
    rt = jnp.transpose(rois_p, (0, 2, 1))

    nms = pl.pallas_call(
        _nms_body,
        grid=(n,),
        in_specs=[
            pl.BlockSpec((1, _NPAD, 4), lambda i: (i, 0, 0)),
            pl.BlockSpec((1, 4, _NPAD), lambda i: (i, 0, 0)),
            pl.BlockSpec((1, _NPAD, 1), lambda i: (i, 0, 0)),
            pl.BlockSpec((1, 1, _NPAD), lambda i: (i, 0, 0)),
        ],
        out_specs=[
            pl.BlockSpec((1, 304, 4), lambda i: (i, 0, 0)),
            pl.BlockSpec((1, 304, 1), lambda i: (i, 0, 0)),
        ],
        out_shape=[
            jax.ShapeDtypeStruct((n, 304, 4), jnp.float32),
            jax.ShapeDtypeStruct((n, 304, 1), jnp.float32),
        ],
        scratch_shapes=[
            pltpu.VMEM((_NPAD, _NPAD), jnp.float32),
        ],
    )
    ob, os_ = nms(rois_p, rt, sp[..., None], sp[:, None, :])
    return jnp.concatenate([ob[:, :_POST, :], os_[:, :_POST, :]], axis=-1)


# speedup vs baseline: 4.0484x; 1.0301x over previous
import numpy as np
import jax
import jax.numpy as jnp
from jax.experimental import pallas as pl
from jax.experimental.pallas import tpu as pltpu

_STRIDE = 16
_SCALES = np.array([128.0, 256.0, 512.0], dtype=np.float32)
_RATIOS = np.array([0.5, 1.0, 2.0], dtype=np.float32)
_PRE = 1000
_POST = 300
_NMS_THR = 0.7
_MIN_SIZE = 16.0
_FH = 50
_PG = 52
_ROWS = 2598
_NPAD = 1024


def _anchor_grids():
    ws = (_SCALES[None, :] * np.sqrt(1.0 / _RATIOS)[:, None]).reshape(-1)
    hs = (_SCALES[None, :] * np.sqrt(_RATIOS)[:, None]).reshape(-1)
    r = np.arange(53, 53 + _ROWS)
    gy, gx = r // _PG, r % _PG
    cx = ((gx - 1) + 0.5) * _STRIDE
    cy = ((gy - 1) + 0.5) * _STRIDE
    out = np.empty((4, _ROWS, 9), dtype=np.float32)
    out[0] = cx[:, None]
    out[1] = cy[:, None]
    out[2] = ws[None, :]
    out[3] = hs[None, :]
    return jnp.asarray(out)


def _head_body(x_ref, w9_ref, bconv_ref, wcls_ref, bcls_ref, wd_ref, bd_ref,
               anc_ref, ms_ref, b0_ref, b1_ref, b2_ref, b3_ref, *, img_h, img_w):
    acc = jnp.zeros((_ROWS, 256), jnp.float32)
    for dy in (-1, 0, 1):
        for dx in (-1, 0, 1):
            k = (dy + 1) * 3 + (dx + 1)
            start = 53 + dy * _PG + dx
            xs = x_ref[0, pl.ds(start, _ROWS), :]
            acc += jnp.dot(xs, w9_ref[k], preferred_element_type=jnp.float32)
    h = jnp.maximum(acc + bconv_ref[:, :], 0.0)

    logits = jnp.dot(h, wcls_ref[:, :], preferred_element_type=jnp.float32) + bcls_ref[:, :]
    score = 1.0 / (1.0 + jnp.exp(-logits))

    dxv = jnp.dot(h, wd_ref[0], preferred_element_type=jnp.float32) + bd_ref[0]
    dyv = jnp.dot(h, wd_ref[1], preferred_element_type=jnp.float32) + bd_ref[1]
    dwv = jnp.dot(h, wd_ref[2], preferred_element_type=jnp.float32) + bd_ref[2]
    dhv = jnp.dot(h, wd_ref[3], preferred_element_type=jnp.float32) + bd_ref[3]
    dwv = jnp.minimum(dwv, 4.135)
    dhv = jnp.minimum(dhv, 4.135)

    cxa = anc_ref[0]
    cya = anc_ref[1]
    wa = anc_ref[2]
    ha = anc_ref[3]
    pcx = dxv * wa + cxa
    pcy = dyv * ha + cya
    pw = jnp.exp(dwv) * wa
    ph = jnp.exp(dhv) * ha

    c0 = jnp.clip(pcx - 0.5 * pw, 0.0, img_h)
    c1 = jnp.clip(pcy - 0.5 * ph, 0.0, img_w)
    c2 = jnp.clip(pcx + 0.5 * pw, 0.0, img_h)
    c3 = jnp.clip(pcy + 0.5 * ph, 0.0, img_w)
    hsz = c2 - c0
    wsz = c3 - c1
    valid = (hsz >= _MIN_SIZE) & (wsz >= _MIN_SIZE) & (score >= 0.0)
    ms_ref[0] = jnp.where(valid, score, -1e9)
    b0_ref[0] = c0
    b1_ref[0] = c1
    b2_ref[0] = c2
    b3_ref[0] = c3


def _nms_body(rc_ref, rt_ref, sc_ref, sr_ref, ob_ref, os_ref, sup_ref):
    rc = rc_ref[0]
    rt = rt_ref[0]
    x1c, y1c, x2c, y2c = rc[:, 0:1], rc[:, 1:2], rc[:, 2:3], rc[:, 3:4]
    x1r, y1r, x2r, y2r = rt[0:1, :], rt[1:2, :], rt[2:3, :], rt[3:4, :]

    area_c = (x2c - x1c) * (y2c - y1c)
    area_r = (x2r - x1r) * (y2r - y1r)
    iw = jnp.maximum(jnp.minimum(x2c, x2r) - jnp.maximum(x1c, x1r), 0.0)
    ih = jnp.maximum(jnp.minimum(y2c, y2r) - jnp.maximum(y1c, y1r), 0.0)
    inter = iw * ih
    iou = inter / (area_c + area_r - inter + 1e-9)

    ii = jax.lax.broadcasted_iota(jnp.int32, (_NPAD, _NPAD), 0)
    jj = jax.lax.broadcasted_iota(jnp.int32, (_NPAD, _NPAD), 1)
    sup_ref[:, :] = jnp.where((iou > _NMS_THR) & (jj > ii), 1.0, 0.0)

    lane128 = jax.lax.broadcasted_iota(jnp.int32, (1, 128), 1)
    keep = jnp.ones((1, _NPAD), jnp.float32)
    for b in range(_NPAD // 128):
        base = b * 128
        kb = keep[:, base:base + 128]

        def inner(i, kb, base=base):
            row = sup_ref[pl.ds(base + i, 1), :][:, base:base + 128]
            ki = jnp.sum(kb * jnp.where(lane128 == i, 1.0, 0.0))
            return kb * (1.0 - ki * row)

        kb = jax.lax.fori_loop(0, 128, inner, kb)
        red = jnp.dot(kb, sup_ref[pl.ds(base, 128), :],
                      preferred_element_type=jnp.float32)
        keep = jnp.where(red >= 0.5, 0.0, keep)

    keep = keep * jnp.where(sr_ref[0] > -1e8, 1.0, 0.0)
    lt = jnp.where(ii <= jj, 1.0, 0.0)
    pos = jnp.dot(keep, lt, preferred_element_type=jnp.float32)
    kk = jax.lax.broadcasted_iota(jnp.int32, (304, _NPAD), 0).astype(jnp.float32) + 1.0
    m = jnp.where(pos == kk, 1.0, 0.0) * keep
    ob_ref[0] = jnp.dot(m, rc, preferred_element_type=jnp.float32)
    os_ref[0] = jnp.dot(m, sc_ref[0], preferred_element_type=jnp.float32)


def _unpad(a):
    full = jnp.pad(a, ((0, 0), (53, 53), (0, 0)))
    g = full.reshape(2, _PG, _PG, 9)[:, 1:51, 1:51, :]
    return g.reshape(2, _FH * _FH * 9)


def kernel(images, features, W_conv, b_conv, W_cls, b_cls, W_loc, b_loc):
    img_h = float(images.shape[2])
    img_w = float(images.shape[3])
    n = features.shape[0]

    xt = jnp.transpose(features, (0, 2, 3, 1))
    xp = jnp.pad(xt, ((0, 0), (1, 1), (1, 1), (0, 0)))
    x = xp.reshape(n, _PG * _PG, 256)

    w9 = jnp.transpose(W_conv, (2, 3, 1, 0)).reshape(9, 256, 256)
    wcls = W_cls[:, :, 0, 0].T
    wloc = W_loc[:, :, 0, 0].T
    wd = jnp.stack([wloc[:, c::4] for c in range(4)])
    bd = jnp.stack([b_loc[c::4] for c in range(4)])[:, None, :]
    anc = _anchor_grids()

    import functools
    head = pl.pallas_call(
        functools.partial(_head_body, img_h=img_h, img_w=img_w),
        grid=(n,),
        in_specs=[
            pl.BlockSpec((1, _PG * _PG, 256), lambda i: (i, 0, 0)),
            pl.BlockSpec((9, 256, 256), lambda i: (0, 0, 0)),
            pl.BlockSpec((1, 256), lambda i: (0, 0)),
            pl.BlockSpec((256, 9), lambda i: (0, 0)),
            pl.BlockSpec((1, 9), lambda i: (0, 0)),
            pl.BlockSpec((4, 256, 9), lambda i: (0, 0, 0)),
            pl.BlockSpec((4, 1, 9), lambda i: (0, 0, 0)),
            pl.BlockSpec((4, _ROWS, 9), lambda i: (0, 0, 0)),
        ],
        out_specs=[pl.BlockSpec((1, _ROWS, 9), lambda i: (i, 0, 0))] * 5,
        out_shape=[jax.ShapeDtypeStruct((n, _ROWS, 9), jnp.float32)] * 5,
    )
    ms, c0, c1, c2, c3 = head(x, w9, b_conv.reshape(1, 256), wcls,
                              b_cls.reshape(1, 9), wd, bd, anc)

    masked = _unpad(ms)
    boxes_flat = jnp.stack([_unpad(c0), _unpad(c1), _unpad(c2), _unpad(c3)], axis=-1)
    top_scores, order = jax.lax.top_k(masked, _PRE)
    top_rois = jnp.take_along_axis(boxes_flat, order[..., None], axis=1)

    rois_p = jnp.pad(top_rois, ((0, 0), (0, _NPAD - _PRE), (0, 0)))
    sp = jnp.pad(top_scores, ((0, 0), (0, _NPAD - _PRE)), constant_values=-1e9)
    rt = jnp.transpose(rois_p, (0, 2, 1))

    nms = pl.pallas_call(
        _nms_body,
        grid=(n,),
        in_specs=[
            pl.BlockSpec((1, _NPAD, 4), lambda i: (i, 0, 0)),
            pl.BlockSpec((1, 4, _NPAD), lambda i: (i, 0, 0)),
            pl.BlockSpec((1, _NPAD, 1), lambda i: (i, 0, 0)),
            pl.BlockSpec((1, 1, _NPAD), lambda i: (i, 0, 0)),
        ],
        out_specs=[
            pl.BlockSpec((1, 304, 4), lambda i: (i, 0, 0)),
            pl.BlockSpec((1, 304, 1), lambda i: (i, 0, 0)),
        ],
        out_shape=[
            jax.ShapeDtypeStruct((n, 304, 4), jnp.float32),
            jax.ShapeDtypeStruct((n, 304, 1), jnp.float32),
        ],
        scratch_shapes=[
            pltpu.VMEM((_NPAD, _NPAD), jnp.float32),
        ],
    )
    ob, os_ = nms(rois_p, rt, sp[..., None], sp[:, None, :])
    return jnp.concatenate([ob[:, :_POST, :], os_[:, :_POST, :]], axis=-1)
